# baseline (speedup 1.0000x reference)
import functools
import numpy as np
import jax
import jax.numpy as jnp
from jax import lax
from jax.experimental import pallas as pl
from jax.experimental.pallas import tpu as pltpu
from jax.experimental.pallas import tpu_sc as plsc

N0 = 100000
N1 = 10000
N2 = 1000
D = 128
EPS = 1e-5
BN_SCALE = np.float32(1.0 / np.sqrt(1.0 + EPS))

NC = 2
NS = 16
NW = NC * NS
CH = 64
NB = 4


def _make_segsum(num_rows, num_segs, ch, nb):
    full = num_rows // ch
    tail = num_rows - full * ch
    maxc = -(-full // NW)
    stripe = 400 if num_segs % 400 == 0 else 200
    zstripes = num_segs // stripe
    zper = -(-zstripes // NS)
    zfull = num_segs // ch
    ztail = num_segs - zfull * ch
    assert full // NW >= nb
    mesh = plsc.VectorSubcoreMesh(core_axis_name="c", subcore_axis_name="s")

    scratch = [
        pltpu.VMEM((maxc * ch,), jnp.int32),
        pltpu.VMEM((nb, ch, D), jnp.float32),
        pltpu.VMEM_SHARED((num_segs, D), jnp.float32),
    ]
    scratch += [pltpu.SemaphoreType.DMA] * (2 * nb)
    if tail:
        scratch.append(pltpu.VMEM((tail,), jnp.int32))

    @functools.partial(
        pl.kernel,
        out_type=jax.ShapeDtypeStruct((NC, num_segs, D), jnp.float32),
        mesh=mesh,
        scratch_types=scratch,
    )
    def seg_kernel(x_hbm, idx_hbm, zeros_hbm, out_hbm, idx_all, rows2, acc,
                   *sems_and_tail):
        c = lax.axis_index("c")
        s = lax.axis_index("s")
        w = s * NC + c
        sems = sems_and_tail[:nb]
        ssems = sems_and_tail[nb:2 * nb]
        maybe_tail = sems_and_tail[2 * nb:]
        lo = (w * full) // NW
        hi = ((w + 1) * full) // NW
        cnt = hi - lo

        def load(j, b):
            return pltpu.make_async_copy(
                x_hbm.at[pl.ds((lo + j) * ch, ch)], rows2.at[b], sems[b])

        def scat(j, b):
            return pltpu.make_async_copy(
                rows2.at[b], acc.at[idx_all.at[pl.ds(j * ch, ch)]], ssems[b])

        for b in range(nb - 1):
            @pl.when(b < cnt)
            def _():
                load(b, b).start()

        pltpu.sync_copy(idx_hbm.at[pl.ds(lo * ch, maxc * ch)], idx_all)

        pltpu.sync_copy(zeros_hbm, rows2.at[nb - 1])
        for t in range(-(-(zfull + (1 if ztail else 0)) // NS)):
            zc = s + NS * t

            @pl.when(zc < zfull)
            def _():
                pltpu.sync_copy(rows2.at[nb - 1], acc.at[pl.ds(zc * ch, ch)])

            if ztail:
                @pl.when(zc == zfull)
                def _():
                    pltpu.sync_copy(rows2.at[nb - 1, pl.ds(0, ztail)],
                                    acc.at[pl.ds(zfull * ch, ztail)])

        plsc.subcore_barrier()

        def body(k, carry):
            for bb in range(nb):
                j = nb * k + bb

                @pl.when(j < cnt)
                def _():
                    load(j, bb).wait()
                    pltpu.async_copy(rows2.at[bb],
                                     acc.at[idx_all.at[pl.ds(j * ch, ch)]],
                                     ssems[bb], add=True)

                    @pl.when(j + nb - 1 < cnt)
                    def _():
                        @pl.when(j >= 1)
                        def _():
                            scat(j - 1, (bb - 1) % nb).wait()

                        load(j + nb - 1, (bb - 1) % nb).start()

            return carry

        lax.fori_loop(0, -(-maxc // nb), body, 0)

        for b in range(nb):
            scat(b, b).wait()

        if tail:
            idxt_v = maybe_tail[0]

            @pl.when(w == NW - 1)
            def _():
                pltpu.sync_copy(idx_hbm.at[pl.ds(full * ch, tail)], idxt_v)
                pltpu.sync_copy(x_hbm.at[pl.ds(full * ch, tail)],
                                rows2.at[0, pl.ds(0, tail)])
                pltpu.sync_copy(rows2.at[0, pl.ds(0, tail)], acc.at[idxt_v],
                                add=True)

        plsc.subcore_barrier()

        for t in range(zper):
            st = s * zper + t

            @pl.when(st < zstripes)
            def _():
                pltpu.sync_copy(acc.at[pl.ds(st * stripe, stripe)],
                                out_hbm.at[c, pl.ds(st * stripe, stripe)])

    return seg_kernel


_segsum1 = _make_segsum(N0, N1, CH, NB)
_segsum2 = _make_segsum(N1, N2, CH, NB)


def _mlp_chain(x, w1, b1, g1, be1, w2, b2, g2, be2):
    h = jnp.dot(x, w1, preferred_element_type=jnp.float32) + b1
    h = jnp.maximum(h * (g1 * BN_SCALE) + be1, 0.0)
    h = jnp.dot(h, w2, preferred_element_type=jnp.float32) + b2
    return jnp.maximum(h * (g2 * BN_SCALE) + be2, 0.0)


RB = 1000


def _stage1_body(p_ref, w1_ref, b1_ref, g_ref, be_ref, w2_ref, b2_ref,
                 bg_ref, bb_ref, h1_ref, p0_ref, p1_ref):
    i = pl.program_id(0)

    @pl.when(i == 0)
    def _():
        p0_ref[...] = jnp.zeros_like(p0_ref)
        p1_ref[...] = jnp.zeros_like(p1_ref)

    x = p_ref[0] + p_ref[1]
    p0_ref[...] += jnp.sum(x, axis=0, keepdims=True)
    h1 = _mlp_chain(x, w1_ref[...], b1_ref[...], g_ref[...], be_ref[...],
                    w2_ref[...], b2_ref[...], bg_ref[...], bb_ref[...])
    h1_ref[...] = h1
    p1_ref[...] += jnp.sum(h1, axis=0, keepdims=True)


_stage1 = pl.pallas_call(
    _stage1_body,
    grid=(N1 // RB,),
    in_specs=[
        pl.BlockSpec((NC, RB, D), lambda i: (0, i, 0)),
        pl.BlockSpec((D, D), lambda i: (0, 0)),
        pl.BlockSpec((1, D), lambda i: (0, 0)),
        pl.BlockSpec((1, D), lambda i: (0, 0)),
        pl.BlockSpec((1, D), lambda i: (0, 0)),
        pl.BlockSpec((D, D), lambda i: (0, 0)),
        pl.BlockSpec((1, D), lambda i: (0, 0)),
        pl.BlockSpec((1, D), lambda i: (0, 0)),
        pl.BlockSpec((1, D), lambda i: (0, 0)),
    ],
    out_specs=[
        pl.BlockSpec((RB, D), lambda i: (i, 0)),
        pl.BlockSpec((1, D), lambda i: (0, 0)),
        pl.BlockSpec((1, D), lambda i: (0, 0)),
    ],
    out_shape=[
        jax.ShapeDtypeStruct((N1, D), jnp.float32),
        jax.ShapeDtypeStruct((1, D), jnp.float32),
        jax.ShapeDtypeStruct((1, D), jnp.float32),
    ],
)


def _stage2_body(p2_ref, w1_ref, b1_ref, g_ref, be_ref, w2_ref, b2_ref,
                 bg_ref, bb_ref, p0w_ref, p0b_ref, p1w_ref, p1b_ref,
                 p2w_ref, p2b_ref, pool0_ref, pool1_ref, out_ref):
    x = p2_ref[0] + p2_ref[1]
    h2 = _mlp_chain(x, w1_ref[...], b1_ref[...], g_ref[...], be_ref[...],
                    w2_ref[...], b2_ref[...], bg_ref[...], bb_ref[...])
    base = (jnp.dot(pool0_ref[...], p0w_ref[...],
                    preferred_element_type=jnp.float32) + p0b_ref[...]
            + jnp.dot(pool1_ref[...], p1w_ref[...],
                      preferred_element_type=jnp.float32) + p1b_ref[...])
    out_ref[...] = (jnp.dot(h2, p2w_ref[...],
                            preferred_element_type=jnp.float32)
                    + p2b_ref[...] + base)


_stage2 = pl.pallas_call(
    _stage2_body,
    out_shape=jax.ShapeDtypeStruct((N2, D), jnp.float32),
)


def kernel(inputs, parent_idx1, parent_idx2,
           mlp1_w1, mlp1_b1, mlp1_g, mlp1_beta, mlp1_w2, mlp1_b2, bn1_g, bn1_b,
           mlp2_w1, mlp2_b1, mlp2_g, mlp2_beta, mlp2_w2, mlp2_b2, bn2_g, bn2_b,
           pred0_w, pred0_b, pred1_w, pred1_b, pred2_w, pred2_b):
    h0 = inputs.reshape(N0, D)
    zeros = jnp.zeros((CH, D), jnp.float32)
    r = lambda v: v.reshape(1, D)

    part1 = _segsum1(h0, parent_idx1, zeros)
    h1, pool0, pool1 = _stage1(part1, mlp1_w1, r(mlp1_b1), r(mlp1_g),
                               r(mlp1_beta), mlp1_w2, r(mlp1_b2), r(bn1_g),
                               r(bn1_b))
    part2 = _segsum2(h1, parent_idx2, zeros)
    logits = _stage2(part2, mlp2_w1, r(mlp2_b1), r(mlp2_g), r(mlp2_beta),
                     mlp2_w2, r(mlp2_b2), r(bn2_g), r(bn2_b),
                     pred0_w, r(pred0_b), pred1_w, r(pred1_b),
                     pred2_w, r(pred2_b), pool0, pool1)
    return logits


# speedup vs baseline: 1.0560x; 1.0560x over previous
import functools
import numpy as np
import jax
import jax.numpy as jnp
from jax import lax
from jax.experimental import pallas as pl
from jax.experimental.pallas import tpu as pltpu
from jax.experimental.pallas import tpu_sc as plsc

N0 = 100000
N1 = 10000
N2 = 1000
D = 128
EPS = 1e-5
BN_SCALE = np.float32(1.0 / np.sqrt(1.0 + EPS))

NC = 2
NS = 16
NW = NC * NS
CH = 64
NB = 4


def _make_segsum(num_rows, num_segs, ch, nb):
    full = num_rows // ch
    tail = num_rows - full * ch
    maxc = -(-full // NW)
    stripe = 400 if num_segs % 400 == 0 else 200
    zstripes = num_segs // stripe
    zper = -(-zstripes // NS)
    zfull = num_segs // ch
    ztail = num_segs - zfull * ch
    assert full // NW >= nb
    mesh = plsc.VectorSubcoreMesh(core_axis_name="c", subcore_axis_name="s")

    scratch = [
        pltpu.VMEM((maxc * ch,), jnp.int32),
        pltpu.VMEM((nb, ch, D), jnp.float32),
        pltpu.VMEM_SHARED((num_segs, D), jnp.float32),
    ]
    scratch += [pltpu.SemaphoreType.DMA] * (2 * nb)
    if tail:
        scratch.append(pltpu.VMEM((tail,), jnp.int32))

    @functools.partial(
        pl.kernel,
        out_type=jax.ShapeDtypeStruct((NC, num_segs, D), jnp.float32),
        mesh=mesh,
        scratch_types=scratch,
    )
    def seg_kernel(x_hbm, idx_hbm, zeros_hbm, out_hbm, idx_all, rows2, acc,
                   *sems_and_tail):
        c = lax.axis_index("c")
        s = lax.axis_index("s")
        w = s * NC + c
        sems = sems_and_tail[:nb]
        ssems = sems_and_tail[nb:2 * nb]
        maybe_tail = sems_and_tail[2 * nb:]
        lo = (w * full) // NW
        hi = ((w + 1) * full) // NW
        cnt = hi - lo

        def load(j, b):
            return pltpu.make_async_copy(
                x_hbm.at[pl.ds((lo + j) * ch, ch)], rows2.at[b], sems[b])

        def scat(j, b):
            return pltpu.make_async_copy(
                rows2.at[b], acc.at[idx_all.at[pl.ds(j * ch, ch)]], ssems[b])

        for b in range(nb - 1):
            @pl.when(b < cnt)
            def _():
                load(b, b).start()

        pltpu.sync_copy(idx_hbm.at[pl.ds(lo * ch, maxc * ch)], idx_all)

        pltpu.sync_copy(zeros_hbm, rows2.at[nb - 1])
        for t in range(-(-(zfull + (1 if ztail else 0)) // NS)):
            zc = s + NS * t

            @pl.when(zc < zfull)
            def _():
                pltpu.sync_copy(rows2.at[nb - 1], acc.at[pl.ds(zc * ch, ch)])

            if ztail:
                @pl.when(zc == zfull)
                def _():
                    pltpu.sync_copy(rows2.at[nb - 1, pl.ds(0, ztail)],
                                    acc.at[pl.ds(zfull * ch, ztail)])

        plsc.subcore_barrier()

        def body(k, carry):
            for bb in range(nb):
                j = nb * k + bb

                @pl.when(j < cnt)
                def _():
                    load(j, bb).wait()
                    pltpu.async_copy(rows2.at[bb],
                                     acc.at[idx_all.at[pl.ds(j * ch, ch)]],
                                     ssems[bb], add=True)

                    @pl.when(j + nb - 1 < cnt)
                    def _():
                        @pl.when(j >= 1)
                        def _():
                            scat(j - 1, (bb - 1) % nb).wait()

                        load(j + nb - 1, (bb - 1) % nb).start()

            return carry

        lax.fori_loop(0, -(-maxc // nb), body, 0)

        for b in range(nb):
            scat(b, b).wait()

        if tail:
            idxt_v = maybe_tail[0]

            @pl.when(w == NW - 1)
            def _():
                pltpu.sync_copy(idx_hbm.at[pl.ds(full * ch, tail)], idxt_v)
                pltpu.sync_copy(x_hbm.at[pl.ds(full * ch, tail)],
                                rows2.at[0, pl.ds(0, tail)])
                pltpu.sync_copy(rows2.at[0, pl.ds(0, tail)], acc.at[idxt_v],
                                add=True)

        plsc.subcore_barrier()

        for t in range(zper):
            st = s * zper + t

            @pl.when(st < zstripes)
            def _():
                pltpu.sync_copy(acc.at[pl.ds(st * stripe, stripe)],
                                out_hbm.at[c, pl.ds(st * stripe, stripe)])

    return seg_kernel


_segsum1 = _make_segsum(N0, N1, CH, NB)
_segsum2 = _make_segsum(N1, N2, CH, NB)


def _mlp_chain(x, w1, b1, g1, be1, w2, b2, g2, be2):
    h = jnp.dot(x, w1, preferred_element_type=jnp.float32) + b1
    h = jnp.maximum(h * (g1 * BN_SCALE) + be1, 0.0)
    h = jnp.dot(h, w2, preferred_element_type=jnp.float32) + b2
    return jnp.maximum(h * (g2 * BN_SCALE) + be2, 0.0)


RB = 5000


def _stage1_body(p_ref, w1_ref, b1_ref, g_ref, be_ref, w2_ref, b2_ref,
                 bg_ref, bb_ref, h1_ref, p0_ref, p1_ref):
    i = pl.program_id(0)

    @pl.when(i == 0)
    def _():
        p0_ref[...] = jnp.zeros_like(p0_ref)
        p1_ref[...] = jnp.zeros_like(p1_ref)

    x = p_ref[0] + p_ref[1]
    p0_ref[...] += jnp.sum(x, axis=0, keepdims=True)
    h1 = _mlp_chain(x, w1_ref[...], b1_ref[...], g_ref[...], be_ref[...],
                    w2_ref[...], b2_ref[...], bg_ref[...], bb_ref[...])
    h1_ref[...] = h1
    p1_ref[...] += jnp.sum(h1, axis=0, keepdims=True)


_stage1 = pl.pallas_call(
    _stage1_body,
    grid=(N1 // RB,),
    in_specs=[
        pl.BlockSpec((NC, RB, D), lambda i: (0, i, 0)),
        pl.BlockSpec((D, D), lambda i: (0, 0)),
        pl.BlockSpec((1, D), lambda i: (0, 0)),
        pl.BlockSpec((1, D), lambda i: (0, 0)),
        pl.BlockSpec((1, D), lambda i: (0, 0)),
        pl.BlockSpec((D, D), lambda i: (0, 0)),
        pl.BlockSpec((1, D), lambda i: (0, 0)),
        pl.BlockSpec((1, D), lambda i: (0, 0)),
        pl.BlockSpec((1, D), lambda i: (0, 0)),
    ],
    out_specs=[
        pl.BlockSpec((RB, D), lambda i: (i, 0)),
        pl.BlockSpec((1, D), lambda i: (0, 0)),
        pl.BlockSpec((1, D), lambda i: (0, 0)),
    ],
    out_shape=[
        jax.ShapeDtypeStruct((N1, D), jnp.float32),
        jax.ShapeDtypeStruct((1, D), jnp.float32),
        jax.ShapeDtypeStruct((1, D), jnp.float32),
    ],
)


def _stage2_body(p2_ref, w1_ref, b1_ref, g_ref, be_ref, w2_ref, b2_ref,
                 bg_ref, bb_ref, p0w_ref, p0b_ref, p1w_ref, p1b_ref,
                 p2w_ref, p2b_ref, pool0_ref, pool1_ref, out_ref):
    x = p2_ref[0] + p2_ref[1]
    h2 = _mlp_chain(x, w1_ref[...], b1_ref[...], g_ref[...], be_ref[...],
                    w2_ref[...], b2_ref[...], bg_ref[...], bb_ref[...])
    base = (jnp.dot(pool0_ref[...], p0w_ref[...],
                    preferred_element_type=jnp.float32) + p0b_ref[...]
            + jnp.dot(pool1_ref[...], p1w_ref[...],
                      preferred_element_type=jnp.float32) + p1b_ref[...])
    out_ref[...] = (jnp.dot(h2, p2w_ref[...],
                            preferred_element_type=jnp.float32)
                    + p2b_ref[...] + base)


_stage2 = pl.pallas_call(
    _stage2_body,
    out_shape=jax.ShapeDtypeStruct((N2, D), jnp.float32),
)


def kernel(inputs, parent_idx1, parent_idx2,
           mlp1_w1, mlp1_b1, mlp1_g, mlp1_beta, mlp1_w2, mlp1_b2, bn1_g, bn1_b,
           mlp2_w1, mlp2_b1, mlp2_g, mlp2_beta, mlp2_w2, mlp2_b2, bn2_g, bn2_b,
           pred0_w, pred0_b, pred1_w, pred1_b, pred2_w, pred2_b):
    h0 = inputs.reshape(N0, D)
    zeros = jnp.zeros((CH, D), jnp.float32)
    r = lambda v: v.reshape(1, D)

    part1 = _segsum1(h0, parent_idx1, zeros)
    h1, pool0, pool1 = _stage1(part1, mlp1_w1, r(mlp1_b1), r(mlp1_g),
                               r(mlp1_beta), mlp1_w2, r(mlp1_b2), r(bn1_g),
                               r(bn1_b))
    part2 = _segsum2(h1, parent_idx2, zeros)
    logits = _stage2(part2, mlp2_w1, r(mlp2_b1), r(mlp2_g), r(mlp2_beta),
                     mlp2_w2, r(mlp2_b2), r(bn2_g), r(bn2_b),
                     pred0_w, r(pred0_b), pred1_w, r(pred1_b),
                     pred2_w, r(pred2_b), pool0, pool1)
    return logits


# speedup vs baseline: 1.0828x; 1.0254x over previous
import functools
import numpy as np
import jax
import jax.numpy as jnp
from jax import lax
from jax.experimental import pallas as pl
from jax.experimental.pallas import tpu as pltpu
from jax.experimental.pallas import tpu_sc as plsc

N0 = 100000
N1 = 10000
N2 = 1000
D = 128
EPS = 1e-5
BN_SCALE = np.float32(1.0 / np.sqrt(1.0 + EPS))

NC = 2
NS = 16
NW = NC * NS
CH = 64
NB = 5


def _make_segsum(num_rows, num_segs, ch, nb):
    full = num_rows // ch
    tail = num_rows - full * ch
    maxc = -(-full // NW)
    stripe = 400 if num_segs % 400 == 0 else 200
    zstripes = num_segs // stripe
    zper = -(-zstripes // NS)
    zfull = num_segs // ch
    ztail = num_segs - zfull * ch
    assert full // NW >= nb
    mesh = plsc.VectorSubcoreMesh(core_axis_name="c", subcore_axis_name="s")

    scratch = [
        pltpu.VMEM((maxc * ch,), jnp.int32),
        pltpu.VMEM((nb, ch, D), jnp.float32),
        pltpu.VMEM_SHARED((num_segs, D), jnp.float32),
    ]
    scratch += [pltpu.SemaphoreType.DMA] * (2 * nb)
    if tail:
        scratch.append(pltpu.VMEM((tail,), jnp.int32))

    @functools.partial(
        pl.kernel,
        out_type=jax.ShapeDtypeStruct((NC, num_segs, D), jnp.float32),
        mesh=mesh,
        scratch_types=scratch,
    )
    def seg_kernel(x_hbm, idx_hbm, zeros_hbm, out_hbm, idx_all, rows2, acc,
                   *sems_and_tail):
        c = lax.axis_index("c")
        s = lax.axis_index("s")
        w = s * NC + c
        sems = sems_and_tail[:nb]
        ssems = sems_and_tail[nb:2 * nb]
        maybe_tail = sems_and_tail[2 * nb:]
        lo = (w * full) // NW
        hi = ((w + 1) * full) // NW
        cnt = hi - lo

        def load(j, b):
            return pltpu.make_async_copy(
                x_hbm.at[pl.ds((lo + j) * ch, ch)], rows2.at[b], sems[b])

        def scat(j, b):
            return pltpu.make_async_copy(
                rows2.at[b], acc.at[idx_all.at[pl.ds(j * ch, ch)]], ssems[b])

        for b in range(nb - 1):
            @pl.when(b < cnt)
            def _():
                load(b, b).start()

        pltpu.sync_copy(idx_hbm.at[pl.ds(lo * ch, maxc * ch)], idx_all)

        pltpu.sync_copy(zeros_hbm, rows2.at[nb - 1])
        for t in range(-(-(zfull + (1 if ztail else 0)) // NS)):
            zc = s + NS * t

            @pl.when(zc < zfull)
            def _():
                pltpu.sync_copy(rows2.at[nb - 1], acc.at[pl.ds(zc * ch, ch)])

            if ztail:
                @pl.when(zc == zfull)
                def _():
                    pltpu.sync_copy(rows2.at[nb - 1, pl.ds(0, ztail)],
                                    acc.at[pl.ds(zfull * ch, ztail)])

        plsc.subcore_barrier()

        def body(k, carry):
            for bb in range(nb):
                j = nb * k + bb

                @pl.when(j < cnt)
                def _():
                    load(j, bb).wait()
                    pltpu.async_copy(rows2.at[bb],
                                     acc.at[idx_all.at[pl.ds(j * ch, ch)]],
                                     ssems[bb], add=True)

                    @pl.when(j + nb - 1 < cnt)
                    def _():
                        @pl.when(j >= 1)
                        def _():
                            scat(j - 1, (bb - 1) % nb).wait()

                        load(j + nb - 1, (bb - 1) % nb).start()

            return carry

        lax.fori_loop(0, -(-maxc // nb), body, 0)

        for b in range(nb):
            scat(b, b).wait()

        if tail:
            idxt_v = maybe_tail[0]

            @pl.when(w == NW - 1)
            def _():
                pltpu.sync_copy(idx_hbm.at[pl.ds(full * ch, tail)], idxt_v)
                pltpu.sync_copy(x_hbm.at[pl.ds(full * ch, tail)],
                                rows2.at[0, pl.ds(0, tail)])
                pltpu.sync_copy(rows2.at[0, pl.ds(0, tail)], acc.at[idxt_v],
                                add=True)

        plsc.subcore_barrier()

        for t in range(zper):
            st = s * zper + t

            @pl.when(st < zstripes)
            def _():
                pltpu.sync_copy(acc.at[pl.ds(st * stripe, stripe)],
                                out_hbm.at[c, pl.ds(st * stripe, stripe)])

    return seg_kernel


_segsum1 = _make_segsum(N0, N1, CH, NB)
_segsum2 = _make_segsum(N1, N2, CH, 4)


def _mlp_chain(x, w1, b1, g1, be1, w2, b2, g2, be2):
    h = jnp.dot(x, w1, preferred_element_type=jnp.float32) + b1
    h = jnp.maximum(h * (g1 * BN_SCALE) + be1, 0.0)
    h = jnp.dot(h, w2, preferred_element_type=jnp.float32) + b2
    return jnp.maximum(h * (g2 * BN_SCALE) + be2, 0.0)


RB = 5000


def _stage1_body(p_ref, w1_ref, b1_ref, g_ref, be_ref, w2_ref, b2_ref,
                 bg_ref, bb_ref, h1_ref, p0_ref, p1_ref):
    i = pl.program_id(0)

    @pl.when(i == 0)
    def _():
        p0_ref[...] = jnp.zeros_like(p0_ref)
        p1_ref[...] = jnp.zeros_like(p1_ref)

    x = p_ref[0] + p_ref[1]
    p0_ref[...] += jnp.sum(x, axis=0, keepdims=True)
    h1 = _mlp_chain(x, w1_ref[...], b1_ref[...], g_ref[...], be_ref[...],
                    w2_ref[...], b2_ref[...], bg_ref[...], bb_ref[...])
    h1_ref[...] = h1
    p1_ref[...] += jnp.sum(h1, axis=0, keepdims=True)


_stage1 = pl.pallas_call(
    _stage1_body,
    grid=(N1 // RB,),
    in_specs=[
        pl.BlockSpec((NC, RB, D), lambda i: (0, i, 0)),
        pl.BlockSpec((D, D), lambda i: (0, 0)),
        pl.BlockSpec((1, D), lambda i: (0, 0)),
        pl.BlockSpec((1, D), lambda i: (0, 0)),
        pl.BlockSpec((1, D), lambda i: (0, 0)),
        pl.BlockSpec((D, D), lambda i: (0, 0)),
        pl.BlockSpec((1, D), lambda i: (0, 0)),
        pl.BlockSpec((1, D), lambda i: (0, 0)),
        pl.BlockSpec((1, D), lambda i: (0, 0)),
    ],
    out_specs=[
        pl.BlockSpec((RB, D), lambda i: (i, 0)),
        pl.BlockSpec((1, D), lambda i: (0, 0)),
        pl.BlockSpec((1, D), lambda i: (0, 0)),
    ],
    out_shape=[
        jax.ShapeDtypeStruct((N1, D), jnp.float32),
        jax.ShapeDtypeStruct((1, D), jnp.float32),
        jax.ShapeDtypeStruct((1, D), jnp.float32),
    ],
)


def _stage2_body(p2_ref, w1_ref, b1_ref, g_ref, be_ref, w2_ref, b2_ref,
                 bg_ref, bb_ref, p0w_ref, p0b_ref, p1w_ref, p1b_ref,
                 p2w_ref, p2b_ref, pool0_ref, pool1_ref, out_ref):
    x = p2_ref[0] + p2_ref[1]
    h2 = _mlp_chain(x, w1_ref[...], b1_ref[...], g_ref[...], be_ref[...],
                    w2_ref[...], b2_ref[...], bg_ref[...], bb_ref[...])
    base = (jnp.dot(pool0_ref[...], p0w_ref[...],
                    preferred_element_type=jnp.float32) + p0b_ref[...]
            + jnp.dot(pool1_ref[...], p1w_ref[...],
                      preferred_element_type=jnp.float32) + p1b_ref[...])
    out_ref[...] = (jnp.dot(h2, p2w_ref[...],
                            preferred_element_type=jnp.float32)
                    + p2b_ref[...] + base)


_stage2 = pl.pallas_call(
    _stage2_body,
    out_shape=jax.ShapeDtypeStruct((N2, D), jnp.float32),
)


def kernel(inputs, parent_idx1, parent_idx2,
           mlp1_w1, mlp1_b1, mlp1_g, mlp1_beta, mlp1_w2, mlp1_b2, bn1_g, bn1_b,
           mlp2_w1, mlp2_b1, mlp2_g, mlp2_beta, mlp2_w2, mlp2_b2, bn2_g, bn2_b,
           pred0_w, pred0_b, pred1_w, pred1_b, pred2_w, pred2_b):
    h0 = inputs.reshape(N0, D)
    zeros = jnp.zeros((CH, D), jnp.float32)
    r = lambda v: v.reshape(1, D)

    part1 = _segsum1(h0, parent_idx1, zeros)
    h1, pool0, pool1 = _stage1(part1, mlp1_w1, r(mlp1_b1), r(mlp1_g),
                               r(mlp1_beta), mlp1_w2, r(mlp1_b2), r(bn1_g),
                               r(bn1_b))
    part2 = _segsum2(h1, parent_idx2, zeros)
    logits = _stage2(part2, mlp2_w1, r(mlp2_b1), r(mlp2_g), r(mlp2_beta),
                     mlp2_w2, r(mlp2_b2), r(bn2_g), r(bn2_b),
                     pred0_w, r(pred0_b), pred1_w, r(pred1_b),
                     pred2_w, r(pred2_b), pool0, pool1)
    return logits
